# baseline (speedup 1.0000x reference)
import functools

import jax
import jax.numpy as jnp
from jax import lax
from jax.experimental import pallas as pl
from jax.experimental.pallas import tpu as pltpu
from jax.experimental.pallas import tpu_sc as plsc

_MESH = plsc.VectorSubcoreMesh(
    core_axis_name="c", subcore_axis_name="s", num_cores=1, num_subcores=1
)


@functools.partial(
    pl.kernel,
    out_type=jax.ShapeDtypeStruct((16,), jnp.float32),
    mesh=_MESH,
    compiler_params=pltpu.CompilerParams(
        needs_layout_passes=False, skip_device_barrier=True
    ),
    scratch_types=[
        pltpu.VMEM((3,), jnp.int32),
        pltpu.VMEM((10, 3), jnp.float32),
        pltpu.VMEM((1, 9), jnp.float32),
        pltpu.VMEM((1,), jnp.float32),
        pltpu.VMEM((16,), jnp.float32),
    ],
)
def _sc_forward(x_hbm, emb_hbm, w_hbm, b_hbm, out_hbm, x_v, emb_v, w_v, b_v, out_v):
    pltpu.sync_copy(x_hbm, x_v)
    pltpu.sync_copy(emb_hbm, emb_v)
    pltpu.sync_copy(w_hbm, w_v)
    pltpu.sync_copy(b_hbm, b_v)
    lane = lax.broadcasted_iota(jnp.int32, (16,), 0)
    dead = jnp.minimum(jnp.maximum(lane - 8, 0), 1)
    live = 1 - dead
    row = jnp.minimum(jnp.maximum(lane - 2, 0), 1) + jnp.minimum(
        jnp.maximum(lane - 5, 0), 1
    )
    col = (lane - 3 * row) * live
    xv = plsc.load_gather(x_v, [row])
    ev = plsc.load_gather(emb_v, [xv, col])
    wg = plsc.load_gather(w_v, [lane * 0, jnp.minimum(lane, 8)])
    bg = plsc.load_gather(b_v, [lane * 0])
    s = jnp.sum(ev * wg * live.astype(jnp.float32))
    y = jnp.full((16,), s, dtype=jnp.float32) + bg
    out_v[...] = 1.0 / (1.0 + jnp.exp(-y))
    pltpu.sync_copy(out_v, out_hbm)


def kernel(x, emb_table, fc_w, fc_b):
    out = _sc_forward(x.astype(jnp.int32), emb_table, fc_w, fc_b)
    return out[:1]


# speedup vs baseline: 1.0548x; 1.0548x over previous
import functools

import jax
import jax.numpy as jnp
from jax import lax
from jax.experimental import pallas as pl
from jax.experimental.pallas import tpu as pltpu
from jax.experimental.pallas import tpu_sc as plsc

_MESH = plsc.VectorSubcoreMesh(
    core_axis_name="c", subcore_axis_name="s", num_cores=1, num_subcores=1
)


@functools.partial(
    pl.kernel,
    out_type=jax.ShapeDtypeStruct((16,), jnp.float32),
    mesh=_MESH,
    compiler_params=pltpu.CompilerParams(
        needs_layout_passes=False, skip_device_barrier=True, use_tc_tiling_on_sc=False
    ),
    scratch_types=[
        pltpu.VMEM((64,), jnp.float32),
        pltpu.VMEM((16,), jnp.float32),
    ],
)
def _sc_forward(buf_hbm, out_hbm, buf_v, out_v):
    pltpu.sync_copy(buf_hbm, buf_v)
    lane = lax.broadcasted_iota(jnp.int32, (16,), 0)
    dead = jnp.minimum(jnp.maximum(lane - 8, 0), 1)
    live = 1 - dead
    row = jnp.minimum(jnp.maximum(lane - 2, 0), 1) + jnp.minimum(
        jnp.maximum(lane - 5, 0), 1
    )
    col = lane - 3 * row
    xv = plsc.load_gather(buf_v, [48 + row]).astype(jnp.int32)
    flat_idx = live * (xv * 3 + col) + dead * 30
    ev = plsc.load_gather(buf_v, [flat_idx])
    wv = buf_v[pl.ds(32, 16)]
    y = jnp.full((16,), jnp.sum(ev * wv), dtype=jnp.float32)
    out_v[...] = 1.0 / (1.0 + jnp.exp(-y))
    pltpu.sync_copy(out_v, out_hbm)


def kernel(x, emb_table, fc_w, fc_b):
    buf = jnp.concatenate(
        [
            emb_table.reshape(-1),
            jnp.array([1.0, 0.0], jnp.float32),
            fc_w.reshape(-1),
            fc_b.reshape(-1),
            jnp.zeros((6,), jnp.float32),
            x.astype(jnp.float32),
            jnp.zeros((13,), jnp.float32),
        ]
    )
    out = _sc_forward(buf)
    return out[:1]
